# baseline (speedup 1.0000x reference)
import functools

import jax
import jax.numpy as jnp
import numpy as np
from jax import lax
from jax.experimental import pallas as pl
from jax.experimental.pallas import tpu as pltpu
from jax.experimental.pallas import tpu_sc as plsc

D = 128
L = 16
NC = 2
NS = 16
NW = NC * NS
CHUNK = 64
K = 5
NCHUNKS = NW * K
VPR = D // L


def _row_base(k, batch):
    return min((k * batch // NCHUNKS) & ~7, batch - CHUNK)


def _make_sc_call(batch, fan_out, scale_val):
    mesh = plsc.VectorSubcoreMesh(core_axis_name="c", subcore_axis_name="s",
                                  num_cores=NC, num_subcores=NS)

    @functools.partial(
        pl.kernel,
        out_type=jax.ShapeDtypeStruct((batch, D), jnp.float32),
        mesh=mesh,
        scratch_types=[
            pltpu.VMEM((K, fan_out, CHUNK), jnp.int32),
            pltpu.VMEM((CHUNK, D), jnp.float32),
            pltpu.VMEM((CHUNK, D), jnp.float32),
            pltpu.VMEM((CHUNK, D), jnp.float32),
            pltpu.VMEM((CHUNK, D), jnp.float32),
            pltpu.SemaphoreType.DMA,
            pltpu.SemaphoreType.DMA,
            pltpu.SemaphoreType.DMA,
            pltpu.SemaphoreType.DMA,
        ],
    )
    def sc_call(feat_hbm, idx_hbm, out_hbm, idx_v, acc0, acc1, st0, st1,
                sg0, sg1, so0, so1):
        c = lax.axis_index("c")
        s = lax.axis_index("s")
        wid = s * NC + c
        base = wid * K
        accs = (acc0, acc1)
        stages = (st0, st1)
        sgs = (sg0, sg1)
        sos = (so0, so1)
        zvec = jnp.zeros((L,), jnp.float32)

        pltpu.sync_copy(idx_hbm.at[pl.ds(base, K)], idx_v)

        def zero(acc):
            def zbody(r, carry):
                for v in range(VPR):
                    acc[r, pl.ds(v * L, L)] = zvec
                return carry
            lax.fori_loop(0, CHUNK, zbody, 0)

        def issue(j, acc, sem):
            def ibody(n, carry):
                pltpu.async_copy(feat_hbm.at[idx_v.at[j, n]], acc, sem,
                                 add=True)
                return carry
            lax.fori_loop(0, fan_out, ibody, 0)

        def drain(j, acc, sem):
            def dbody(n, carry):
                pltpu.make_async_copy(feat_hbm.at[idx_v.at[j, n]], acc,
                                      sem).wait()
                return carry
            lax.fori_loop(0, fan_out, dbody, 0)

        def scale_to(acc, stage):
            def sbody(r, carry):
                for v in range(VPR):
                    stage[r, pl.ds(v * L, L)] = (
                        acc[r, pl.ds(v * L, L)] * scale_val)
                return carry
            lax.fori_loop(0, CHUNK, sbody, 0)

        def out_copy(j, stage, sem):
            k = base + j
            rb = jnp.minimum((k * batch // NCHUNKS) // 8 * 8, batch - CHUNK)
            return pltpu.make_async_copy(stage, out_hbm.at[pl.ds(rb, CHUNK)],
                                         sem)

        for slot in range(2):
            zero(accs[slot])
            issue(slot, accs[slot], sgs[slot])

        for j in range(K):
            p = j % 2
            drain(j, accs[p], sgs[p])
            if j >= 2:
                out_copy(j - 2, stages[p], sos[p]).wait()
            scale_to(accs[p], stages[p])
            out_copy(j, stages[p], sos[p]).start()
            if j + 2 < K:
                zero(accs[p])
                issue(j + 2, accs[p], sgs[p])

        for j in (K - 2, K - 1):
            out_copy(j, stages[j % 2], sos[j % 2]).wait()

    return sc_call


def kernel(features, neigh_idx, num_sample):
    n_nodes, d = features.shape
    batch, fan_out = neigh_idx.shape
    assert d == D
    idx = neigh_idx.astype(jnp.int32)
    row_ids = np.concatenate(
        [np.arange(_row_base(k, batch), _row_base(k, batch) + CHUNK)
         for k in range(NCHUNKS)])
    idx3 = jnp.take(idx, jnp.asarray(row_ids), axis=0).reshape(
        NCHUNKS, CHUNK, fan_out).transpose(0, 2, 1)
    scale = jnp.float32(1.0 / fan_out)
    sc_call = _make_sc_call(batch, fan_out, scale)
    return sc_call(features, idx3)


# speedup vs baseline: 1.2023x; 1.2023x over previous
import functools

import jax
import jax.numpy as jnp
from jax import lax
from jax.experimental import pallas as pl
from jax.experimental.pallas import tpu as pltpu
from jax.experimental.pallas import tpu_sc as plsc

D = 128
L = 16
NC = 2
NS = 16
NW = NC * NS
CHUNK = 64
K = 5
NCHUNKS = NW * K
BATCH_PAD = NCHUNKS * CHUNK
VPR = D // L


def _make_sc_call(batch, fan_out, scale_val):
    nfull = batch // CHUNK
    rem = batch - nfull * CHUNK
    mesh = plsc.VectorSubcoreMesh(core_axis_name="c", subcore_axis_name="s",
                                  num_cores=NC, num_subcores=NS)

    @functools.partial(
        pl.kernel,
        out_type=jax.ShapeDtypeStruct((batch, D), jnp.float32),
        mesh=mesh,
        scratch_types=[
            pltpu.VMEM((K, fan_out, CHUNK), jnp.int32),
            pltpu.VMEM((CHUNK, D), jnp.float32),
            pltpu.VMEM((CHUNK, D), jnp.float32),
            pltpu.VMEM((CHUNK, D), jnp.float32),
            pltpu.VMEM((CHUNK, D), jnp.float32),
            pltpu.SemaphoreType.DMA,
            pltpu.SemaphoreType.DMA,
            pltpu.SemaphoreType.DMA,
            pltpu.SemaphoreType.DMA,
        ],
    )
    def sc_call(feat_hbm, idx_hbm, out_hbm, idx_v, acc0, acc1, st0, st1,
                sg0, sg1, so0, so1):
        c = lax.axis_index("c")
        s = lax.axis_index("s")
        wid = s * NC + c
        base = wid * K
        accs = (acc0, acc1)
        stages = (st0, st1)
        sgs = (sg0, sg1)
        sos = (so0, so1)
        zvec = jnp.zeros((L,), jnp.float32)

        pltpu.sync_copy(idx_hbm.at[pl.ds(base, K)], idx_v)

        def zero(acc):
            def zbody(r, carry):
                for v in range(VPR):
                    acc[r, pl.ds(v * L, L)] = zvec
                return carry
            lax.fori_loop(0, CHUNK, zbody, 0)

        def issue(j, acc, sem):
            def ibody(n, carry):
                pltpu.async_copy(feat_hbm.at[idx_v.at[j, n]], acc, sem,
                                 add=True)
                return carry
            lax.fori_loop(0, fan_out, ibody, 0)

        def drain(j, acc, sem):
            def dbody(n, carry):
                pltpu.make_async_copy(feat_hbm.at[idx_v.at[j, n]], acc,
                                      sem).wait()
                return carry
            lax.fori_loop(0, fan_out, dbody, 0)

        def scale_to(acc, stage):
            def sbody(r, carry):
                for v in range(VPR):
                    stage[r, pl.ds(v * L, L)] = (
                        acc[r, pl.ds(v * L, L)] * scale_val)
                return carry
            lax.fori_loop(0, CHUNK, sbody, 0)

        def full_copy(j, stage, sem):
            return pltpu.make_async_copy(
                stage, out_hbm.at[pl.ds((base + j) * CHUNK, CHUNK)], sem)

        def part_copy(stage, sem):
            return pltpu.make_async_copy(
                stage.at[pl.ds(0, rem)],
                out_hbm.at[pl.ds(nfull * CHUNK, rem)], sem)

        def out_start(j, stage, sem):
            k = base + j

            @pl.when(k < nfull)
            def _():
                full_copy(j, stage, sem).start()

            if rem:
                @pl.when(k == nfull)
                def _():
                    part_copy(stage, sem).start()

        def out_wait(j, stage, sem):
            k = base + j

            @pl.when(k < nfull)
            def _():
                full_copy(j, stage, sem).wait()

            if rem:
                @pl.when(k == nfull)
                def _():
                    part_copy(stage, sem).wait()

        def live(j):
            return (base + j) < nfull + (1 if rem else 0)

        for slot in range(2):
            @pl.when(live(slot))
            def _():
                zero(accs[slot])
                issue(slot, accs[slot], sgs[slot])

        for j in range(K):
            p = j % 2

            @pl.when(live(j))
            def _():
                drain(j, accs[p], sgs[p])

            if j >= 2:
                out_wait(j - 2, stages[p], sos[p])

            @pl.when(live(j))
            def _():
                scale_to(accs[p], stages[p])

            out_start(j, stages[p], sos[p])

            if j + 2 < K:
                @pl.when(live(j + 2))
                def _():
                    zero(accs[p])
                    issue(j + 2, accs[p], sgs[p])

        for j in (K - 2, K - 1):
            out_wait(j, stages[j % 2], sos[j % 2])

    return sc_call


def kernel(features, neigh_idx, num_sample):
    n_nodes, d = features.shape
    batch, fan_out = neigh_idx.shape
    assert d == D
    idx = neigh_idx.astype(jnp.int32)
    pad = BATCH_PAD - batch
    if pad:
        fill = (jnp.arange(pad * fan_out, dtype=jnp.int32) % n_nodes
                ).reshape(pad, fan_out)
        idx = jnp.concatenate([idx, fill], axis=0)
    idx3 = idx.reshape(NCHUNKS, CHUNK, fan_out).transpose(0, 2, 1)
    scale = jnp.float32(1.0 / fan_out)
    sc_call = _make_sc_call(batch, fan_out, scale)
    return sc_call(features, idx3)


# speedup vs baseline: 1.2028x; 1.0004x over previous
import functools

import jax
import jax.numpy as jnp
from jax import lax
from jax.experimental import pallas as pl
from jax.experimental.pallas import tpu as pltpu
from jax.experimental.pallas import tpu_sc as plsc

D = 128
L = 16
NC = 2
NS = 16
NW = NC * NS
CHUNK = 64
K = 5
NSLOT = 3
NCHUNKS = NW * K
BATCH_PAD = NCHUNKS * CHUNK
VPR = D // L


def _make_sc_call(batch, fan_out, scale_val):
    nfull = batch // CHUNK
    rem = batch - nfull * CHUNK
    nlive = nfull + (1 if rem else 0)
    mesh = plsc.VectorSubcoreMesh(core_axis_name="c", subcore_axis_name="s",
                                  num_cores=NC, num_subcores=NS)

    @functools.partial(
        pl.kernel,
        out_type=jax.ShapeDtypeStruct((batch, D), jnp.float32),
        mesh=mesh,
        scratch_types=(
            [pltpu.VMEM((K, fan_out, CHUNK), jnp.int32)]
            + [pltpu.VMEM((CHUNK, D), jnp.float32)] * NSLOT
            + [pltpu.VMEM((CHUNK, D), jnp.float32)] * NSLOT
            + [pltpu.SemaphoreType.DMA] * NSLOT
            + [pltpu.SemaphoreType.DMA] * NSLOT
        ),
    )
    def sc_call(feat_hbm, idx_hbm, out_hbm, idx_v, *bufs):
        accs = bufs[:NSLOT]
        stages = bufs[NSLOT:2 * NSLOT]
        sgs = bufs[2 * NSLOT:3 * NSLOT]
        sos = bufs[3 * NSLOT:4 * NSLOT]
        c = lax.axis_index("c")
        s = lax.axis_index("s")
        wid = s * NC + c
        base = wid * K
        zvec = jnp.zeros((L,), jnp.float32)

        pltpu.sync_copy(idx_hbm.at[pl.ds(base, K)], idx_v)

        def zero(acc):
            def zbody(r, carry):
                for v in range(VPR):
                    acc[r, pl.ds(v * L, L)] = zvec
                return carry
            lax.fori_loop(0, CHUNK, zbody, 0)

        def issue(j, acc, sem):
            def ibody(n, carry):
                pltpu.async_copy(feat_hbm.at[idx_v.at[j, n]], acc, sem,
                                 add=True)
                return carry
            lax.fori_loop(0, fan_out, ibody, 0)

        def drain(j, acc, sem):
            def dbody(n, carry):
                pltpu.make_async_copy(feat_hbm.at[idx_v.at[j, n]], acc,
                                      sem).wait()
                return carry
            lax.fori_loop(0, fan_out, dbody, 0)

        def scale_to(acc, stage):
            def sbody(r, carry):
                for v in range(VPR):
                    stage[r, pl.ds(v * L, L)] = (
                        acc[r, pl.ds(v * L, L)] * scale_val)
                return carry
            lax.fori_loop(0, CHUNK, sbody, 0)

        def full_copy(j, stage, sem):
            return pltpu.make_async_copy(
                stage, out_hbm.at[pl.ds((base + j) * CHUNK, CHUNK)], sem)

        def part_copy(stage, sem):
            return pltpu.make_async_copy(
                stage.at[pl.ds(0, rem)],
                out_hbm.at[pl.ds(nfull * CHUNK, rem)], sem)

        def out_start(j, stage, sem):
            k = base + j

            @pl.when(k < nfull)
            def _():
                full_copy(j, stage, sem).start()

            if rem:
                @pl.when(k == nfull)
                def _():
                    part_copy(stage, sem).start()

        def out_wait(j, stage, sem):
            k = base + j

            @pl.when(k < nfull)
            def _():
                full_copy(j, stage, sem).wait()

            if rem:
                @pl.when(k == nfull)
                def _():
                    part_copy(stage, sem).wait()

        def live(j):
            return (base + j) < nlive

        for slot in range(NSLOT):
            @pl.when(live(slot))
            def _():
                zero(accs[slot])
                issue(slot, accs[slot], sgs[slot])

        for j in range(K):
            p = j % NSLOT

            @pl.when(live(j))
            def _():
                drain(j, accs[p], sgs[p])

            if j >= NSLOT:
                out_wait(j - NSLOT, stages[p], sos[p])

            @pl.when(live(j))
            def _():
                scale_to(accs[p], stages[p])

            out_start(j, stages[p], sos[p])

            if j + NSLOT < K:
                @pl.when(live(j + NSLOT))
                def _():
                    zero(accs[p])
                    issue(j + NSLOT, accs[p], sgs[p])

        for j in range(max(K - NSLOT, 0), K):
            out_wait(j, stages[j % NSLOT], sos[j % NSLOT])

    return sc_call


def kernel(features, neigh_idx, num_sample):
    n_nodes, d = features.shape
    batch, fan_out = neigh_idx.shape
    assert d == D
    idx = neigh_idx.astype(jnp.int32)
    pad = BATCH_PAD - batch
    if pad:
        fill = (jnp.arange(pad * fan_out, dtype=jnp.int32) % n_nodes
                ).reshape(pad, fan_out)
        idx = jnp.concatenate([idx, fill], axis=0)
    idx3 = idx.reshape(NCHUNKS, CHUNK, fan_out).transpose(0, 2, 1)
    scale = jnp.float32(1.0 / fan_out)
    sc_call = _make_sc_call(batch, fan_out, scale)
    return sc_call(features, idx3)
